# Initial kernel scaffold; baseline (speedup 1.0000x reference)
import jax
import jax.numpy as jnp
from jax.experimental import pallas as pl


def kernel(features, geometry, lattice, W, mu, max_radius):
    raise NotImplementedError("write your pallas kernel here")


# speedup vs baseline: 19.4713x; 19.4713x over previous
import jax
import jax.numpy as jnp
from jax.experimental import pallas as pl
from jax.experimental.pallas import tpu as pltpu

_GAMMA = 4.0
_NB = 10
_P = 512
_F = 32


def _conv_kernel(params_ref, mu_ref, feat_ref, geom_a_ref, geom_b_ref,
                 wt_ref, out_ref):
    f = feat_ref[0]
    wt = wt_ref[...]
    g = jnp.dot(f, wt, preferred_element_type=jnp.float32)

    d2 = jnp.zeros((_P, _P), jnp.float32)
    for c in range(3):
        b_row = geom_b_ref[0, c:c + 1, :]
        a_col = geom_a_ref[0, :, c:c + 1]
        diff = b_row - a_col
        lc = params_ref[c]
        wrapped = diff - lc * jnp.round(diff * (1.0 / lc))
        d2 = d2 + wrapped * wrapped
    d = jnp.sqrt(d2 + 1e-12)
    rmax = params_ref[3]
    mask = (d <= rmax).astype(jnp.float32)

    acc = jnp.zeros((_P, _F), jnp.float32)
    for k in range(_NB):
        mk = mu_ref[k]
        t = d - mk
        phi = jnp.exp(-_GAMMA * (t * t)) * mask
        gk = g[:, k * _F:(k + 1) * _F]
        acc = acc + jnp.dot(phi, gk, preferred_element_type=jnp.float32)
    out_ref[0] = acc


def kernel(features, geometry, lattice, W, mu, max_radius):
    B = features.shape[0]
    geometry = geometry.astype(jnp.float32)
    geom_t = geometry.transpose(0, 2, 1)
    wt = W.transpose(2, 0, 1).reshape(_F, _NB * _F)
    params = jnp.stack([lattice[0, 0], lattice[1, 1], lattice[2, 2],
                        jnp.asarray(max_radius, jnp.float32)])
    return pl.pallas_call(
        _conv_kernel,
        grid=(B,),
        in_specs=[
            pl.BlockSpec(memory_space=pltpu.SMEM),
            pl.BlockSpec(memory_space=pltpu.SMEM),
            pl.BlockSpec((1, _P, _F), lambda z: (z, 0, 0)),
            pl.BlockSpec((1, _P, 3), lambda z: (z, 0, 0)),
            pl.BlockSpec((1, 3, _P), lambda z: (z, 0, 0)),
            pl.BlockSpec((_F, _NB * _F), lambda z: (0, 0)),
        ],
        out_specs=pl.BlockSpec((1, _P, _F), lambda z: (z, 0, 0)),
        out_shape=jax.ShapeDtypeStruct((B, _P, _F), jnp.float32),
    )(params, mu.astype(jnp.float32), features, geometry, geom_t, wt)
